# baseline (speedup 1.0000x reference)
import jax
import jax.numpy as jnp
from jax.experimental import pallas as pl

E = 800000
K = 64
BE = 16384
BF = BE * K
BN = BF // 128


def _block_kernel(gu_ref, gi_ref, gut_ref, git_ref, o0_ref, o1_ref):
    a = gu_ref[...].reshape(BN, 128)
    b = gi_ref[...].reshape(BN, 128)
    c = gut_ref[...].reshape(BN, 128)
    d = git_ref[...].reshape(BN, 128)
    p = a * b + c * d
    t = p.T
    o0_ref[...] = jnp.sum(t[:K, :], axis=0)
    o1_ref[...] = jnp.sum(t[K:, :], axis=0)


def kernel(gu, gi, gut, git):
    grid = ((E + BE - 1) // BE,)
    in_spec = pl.BlockSpec((BF,), lambda i: (i,))
    out_spec = pl.BlockSpec((BN,), lambda i: (i,))
    o0, o1 = pl.pallas_call(
        _block_kernel,
        grid=grid,
        in_specs=[in_spec] * 4,
        out_specs=[out_spec, out_spec],
        out_shape=[
            jax.ShapeDtypeStruct((E // 2,), jnp.float32),
            jax.ShapeDtypeStruct((E // 2,), jnp.float32),
        ],
    )(gu.reshape(-1), gi.reshape(-1), gut.reshape(-1), git.reshape(-1))
    return jnp.stack([o0, o1], axis=1).reshape(E)


# speedup vs baseline: 1.3939x; 1.3939x over previous
import jax
import jax.numpy as jnp
from jax.experimental import pallas as pl

E = 800000
K = 64
BE = 8192


def _block_kernel(gu_ref, gi_ref, gut_ref, git_ref, out_ref):
    p = gu_ref[...] * gi_ref[...] + gut_ref[...] * git_ref[...]
    out_ref[...] = jnp.sum(p.T, axis=0)


def kernel(gu, gi, gut, git):
    grid = ((E + BE - 1) // BE,)
    in_spec = pl.BlockSpec((BE, K), lambda i: (i, 0))
    out = pl.pallas_call(
        _block_kernel,
        grid=grid,
        in_specs=[in_spec, in_spec, in_spec, in_spec],
        out_specs=pl.BlockSpec((BE,), lambda i: (i,)),
        out_shape=jax.ShapeDtypeStruct((E,), jnp.float32),
    )(gu, gi, gut, git)
    return out


# speedup vs baseline: 9.1266x; 6.5474x over previous
import jax
import jax.numpy as jnp
from jax.experimental import pallas as pl

E = 800000
K = 64
BW = 8192


def _block_kernel(gu_ref, gi_ref, gut_ref, git_ref, out_ref):
    p = gu_ref[...] * gi_ref[...] + gut_ref[...] * git_ref[...]
    out_ref[...] = jnp.sum(p, axis=0)


def kernel(gu, gi, gut, git):
    grid = ((E + BW - 1) // BW,)
    in_spec = pl.BlockSpec((K, BW), lambda i: (0, i))
    out = pl.pallas_call(
        _block_kernel,
        grid=grid,
        in_specs=[in_spec, in_spec, in_spec, in_spec],
        out_specs=pl.BlockSpec((BW,), lambda i: (i,)),
        out_shape=jax.ShapeDtypeStruct((E,), jnp.float32),
    )(gu.T, gi.T, gut.T, git.T)
    return out
